# baseline (speedup 1.0000x reference)
import functools

import jax
import jax.numpy as jnp
from jax import lax
from jax.experimental import pallas as pl
from jax.experimental.pallas import tpu as pltpu
from jax.experimental.pallas import tpu_sc as plsc

B, S, D = 4, 4096, 1024
NUM_EMB = 4

ROWS = B * S
BLK = 1024
STAGE_ROWS = 8


def _sc_body(idx_hbm, table_hbm, out_hbm, idx_v, buf, gsem):
    wid = lax.axis_index("s") * 2 + lax.axis_index("c")

    @pl.when(wid == 0)
    def _stage():
        pltpu.sync_copy(idx_hbm, idx_v)
        pltpu.async_copy(table_hbm.at[idx_v], buf, gsem).wait()
        pltpu.sync_copy(buf, out_hbm)


@functools.partial(
    pl.kernel,
    out_type=jax.ShapeDtypeStruct((STAGE_ROWS, D), jnp.float32),
    mesh=plsc.VectorSubcoreMesh(core_axis_name="c", subcore_axis_name="s"),
    scratch_types=[
        pltpu.VMEM((STAGE_ROWS,), jnp.int32),
        pltpu.VMEM((STAGE_ROWS, D), jnp.float32),
        pltpu.SemaphoreType.DMA,
    ],
)
def _sc_lookup(idx_hbm, table_hbm, out_hbm, idx_v, buf, gsem):
    _sc_body(idx_hbm, table_hbm, out_hbm, idx_v, buf, gsem)


def _bcast_kernel(row_ref, out_ref):
    out_ref[...] = jnp.broadcast_to(row_ref[0:1, :], out_ref.shape)


def kernel(x, modality_index, emb_table):
    del x
    idx_vec = jnp.full((STAGE_ROWS,), modality_index, dtype=jnp.int32)
    staged = _sc_lookup(idx_vec, emb_table)

    out = pl.pallas_call(
        _bcast_kernel,
        grid=(ROWS // BLK,),
        in_specs=[pl.BlockSpec((STAGE_ROWS, D), lambda i: (0, 0))],
        out_specs=pl.BlockSpec((BLK, D), lambda i: (i, 0)),
        out_shape=jax.ShapeDtypeStruct((ROWS, D), jnp.float32),
    )(staged)
    return out.reshape(B, S, D)
